# Initial kernel scaffold; baseline (speedup 1.0000x reference)
import jax
import jax.numpy as jnp
from jax.experimental import pallas as pl


def kernel(F_S):
    raise NotImplementedError("write your pallas kernel here")


# speedup vs baseline: 16.4469x; 16.4469x over previous
import functools

import jax
import jax.numpy as jnp
from jax.experimental import pallas as pl
from jax.experimental.pallas import tpu as pltpu

NBINS = 256
EPSV = 1e-12
CH = 1024


def _enhance_body(f_hbm, out_hbm, f_buf, ohhi_ref, ohlo_ref, in_sem, out_sem):
    b = pl.program_id(0)
    C, P = f_buf.shape
    nch = P // CH
    fP = jnp.float32(P)

    cp_in = pltpu.make_async_copy(f_hbm.at[b], f_buf, in_sem)
    cp_in.start()
    cp_in.wait()

    def phase_a(i, acc):
        fc = f_buf[:, pl.ds(i * CH, CH)]
        part = acc
        for j in range(CH // 128):
            part = part + fc[:, j * 128:(j + 1) * 128]
        return part

    acc = jax.lax.fori_loop(0, nch, phase_a, jnp.zeros((C, 128), jnp.float32))
    a_rep = jnp.dot(acc, jnp.ones((128, 128), jnp.float32),
                    precision=jax.lax.Precision.HIGHEST,
                    preferred_element_type=jnp.float32) / fP
    na = jnp.maximum(
        jnp.sqrt(jnp.sum(a_rep * a_rep, axis=0, keepdims=True)), EPSV)
    an128 = a_rep / na
    an_rep = jnp.tile(an128, (1, CH // 128))

    ioc = jax.lax.broadcasted_iota(jnp.int32, (16, CH), 0)

    def phase_b(i, hist):
        ds = pl.ds(i * CH, CH)
        fc = f_buf[:, ds]
        dotc = jnp.sum(fc * an_rep, axis=0, keepdims=True)
        ssq = jnp.sum(fc * fc, axis=0, keepdims=True)
        npx = jnp.maximum(jnp.sqrt(ssq), EPSV)
        cos = dotc / npx
        qi = (cos * 255.0).astype(jnp.int32)
        q = (qi + 256) & 255
        ohhi = jnp.where((q >> 4) == ioc, 1.0, 0.0)
        ohlo = jnp.where((q & 15) == ioc, 1.0, 0.0)
        ohhi_ref[:, ds] = ohhi
        ohlo_ref[:, ds] = ohlo
        return hist + jax.lax.dot_general(
            ohhi, ohlo, (((1,), (1,)), ((), ())),
            preferred_element_type=jnp.float32)

    hist = jax.lax.fori_loop(0, nch, phase_b,
                             jnp.zeros((16, 16), jnp.float32))

    r16 = jax.lax.broadcasted_iota(jnp.int32, (16, 16), 0)
    c16 = jax.lax.broadcasted_iota(jnp.int32, (16, 16), 1)
    upper = jnp.where(r16 <= c16, 1.0, 0.0)
    lstrict = jnp.where(c16 < r16, 1.0, 0.0)
    cdf_lo = jnp.dot(hist, upper, precision=jax.lax.Precision.HIGHEST,
                     preferred_element_type=jnp.float32)
    rowsum = jnp.sum(hist, axis=1, keepdims=True)
    offs = jnp.dot(lstrict, rowsum, precision=jax.lax.Precision.HIGHEST,
                   preferred_element_type=jnp.float32)
    cdf = cdf_lo + offs
    masked = jnp.where(hist > 0.0, cdf, fP + 1.0)
    cmin = jnp.min(jnp.min(masked, axis=1, keepdims=True),
                   axis=0, keepdims=True)
    denom = jnp.maximum(fP - cmin, 1.0)
    lut = jnp.clip(jnp.round((cdf - cmin) * (255.0 / denom)), 0.0, 255.0)

    def phase_d(i, carry):
        ds = pl.ds(i * CH, CH)
        t = jnp.dot(lut, ohlo_ref[:, ds],
                    preferred_element_type=jnp.float32)
        tt = jnp.sum(ohhi_ref[:, ds] * t, axis=0, keepdims=True)
        f_buf[:, ds] = f_buf[:, ds] * (tt * jnp.float32(1.0 / 255.0))
        return carry

    jax.lax.fori_loop(0, nch, phase_d, 0)

    cp_out = pltpu.make_async_copy(f_buf, out_hbm.at[b], out_sem)
    cp_out.start()
    cp_out.wait()


@functools.partial(jax.jit, static_argnames=())
def kernel(F_S):
    B, C, H, W = F_S.shape
    P = H * W
    f2 = F_S.reshape(B, C, P)
    out = pl.pallas_call(
        _enhance_body,
        grid=(B,),
        in_specs=[pl.BlockSpec(memory_space=pl.ANY)],
        out_specs=pl.BlockSpec(memory_space=pl.ANY),
        out_shape=jax.ShapeDtypeStruct((B, C, P), jnp.float32),
        scratch_shapes=[
            pltpu.VMEM((C, P), jnp.float32),
            pltpu.VMEM((16, P), jnp.float32),
            pltpu.VMEM((16, P), jnp.float32),
            pltpu.SemaphoreType.DMA,
            pltpu.SemaphoreType.DMA,
        ],
        compiler_params=pltpu.CompilerParams(
            dimension_semantics=("parallel",),
            vmem_limit_bytes=100 * 1024 * 1024,
        ),
    )(f2)
    return out.reshape(B, C, H, W)


# speedup vs baseline: 27.1800x; 1.6526x over previous
import jax
import jax.numpy as jnp
from jax.experimental import pallas as pl
from jax.experimental.pallas import tpu as pltpu

EPSV = 1e-12
HC = 8
HIGH = jax.lax.Precision.HIGHEST


def _enhance_body(f_hbm, out_hbm, f_buf, q_ref, in_sem, out_sem):
    b = pl.program_id(0)
    C, H, W = f_buf.shape
    nch = H // HC
    fP = jnp.float32(H * W)

    cp_in = pltpu.make_async_copy(f_hbm.at[b], f_buf, in_sem)
    cp_in.start()
    cp_in.wait()

    def phase_a(i, acc):
        fc = f_buf[:, pl.ds(i * HC, HC), :]
        return acc + jnp.sum(fc, axis=1)

    acc = jax.lax.fori_loop(0, nch, phase_a, jnp.zeros((C, W), jnp.float32))
    a_rep = jnp.dot(acc, jnp.ones((W, W), jnp.float32),
                    precision=HIGH, preferred_element_type=jnp.float32) / fP
    na = jnp.maximum(
        jnp.sqrt(jnp.sum(a_rep * a_rep, axis=0, keepdims=True)), EPSV)
    an3 = jnp.broadcast_to((a_rep / na)[:, None, :], (C, HC, W))

    kpat = jax.lax.broadcasted_iota(jnp.int32, (16 * HC, W), 0) >> 3

    def onehots(q):
        hi_t = jnp.tile(q >> 4, (16, 1))
        lo_t = jnp.tile(q & 15, (16, 1))
        oh_hi = jnp.where(hi_t == kpat, 1.0, 0.0)
        oh_lo = jnp.where(lo_t == kpat, 1.0, 0.0)
        return oh_hi, oh_lo

    def phase_b(i, m128):
        ds = pl.ds(i * HC, HC)
        fc = f_buf[:, ds, :]
        dotc = jnp.sum(fc * an3, axis=0)
        ssq = jnp.sum(fc * fc, axis=0)
        npx = jnp.maximum(jnp.sqrt(ssq), EPSV)
        cos = dotc / npx
        qi = (cos * 255.0).astype(jnp.int32)
        q = (qi + 256) & 255
        q_ref[ds, :] = q.astype(jnp.float32)
        oh_hi, oh_lo = onehots(q)
        return m128 + jax.lax.dot_general(
            oh_hi, oh_lo, (((1,), (1,)), ((), ())),
            preferred_element_type=jnp.float32)

    m128 = jax.lax.fori_loop(0, nch, phase_b,
                             jnp.zeros((16 * HC, 16 * HC), jnp.float32))

    i128r = jax.lax.broadcasted_iota(jnp.int32, (16 * HC, 16 * HC), 0)
    i128c = jax.lax.broadcasted_iota(jnp.int32, (16 * HC, 16 * HC), 1)
    dmask = jnp.where((i128r & 7) == (i128c & 7), 1.0, 0.0)
    s16r = jax.lax.broadcasted_iota(jnp.int32, (16, 16 * HC), 0)
    s16c = jax.lax.broadcasted_iota(jnp.int32, (16, 16 * HC), 1)
    smat = jnp.where(s16r == (s16c >> 3), 1.0, 0.0)
    hist = jnp.dot(jnp.dot(smat, m128 * dmask, precision=HIGH,
                           preferred_element_type=jnp.float32),
                   smat.T, precision=HIGH,
                   preferred_element_type=jnp.float32)

    r16 = jax.lax.broadcasted_iota(jnp.int32, (16, 16), 0)
    c16 = jax.lax.broadcasted_iota(jnp.int32, (16, 16), 1)
    upper = jnp.where(r16 <= c16, 1.0, 0.0)
    lstrict = jnp.where(c16 < r16, 1.0, 0.0)
    cdf_lo = jnp.dot(hist, upper, precision=HIGH,
                     preferred_element_type=jnp.float32)
    rowsum = jnp.sum(hist, axis=1, keepdims=True)
    offs = jnp.dot(lstrict, rowsum, precision=HIGH,
                   preferred_element_type=jnp.float32)
    cdf = cdf_lo + offs
    masked = jnp.where(hist > 0.0, cdf, fP + 1.0)
    cmin = jnp.min(jnp.min(masked, axis=1, keepdims=True),
                   axis=0, keepdims=True)
    denom = jnp.maximum(fP - cmin, 1.0)
    lut = jnp.clip(jnp.round((cdf - cmin) * (255.0 / denom)), 0.0, 255.0)

    lut_d = jnp.dot(jnp.dot(smat.T, lut,
                            preferred_element_type=jnp.float32),
                    smat, preferred_element_type=jnp.float32) * dmask

    def phase_d(i, carry):
        ds = pl.ds(i * HC, HC)
        q = q_ref[ds, :].astype(jnp.int32)
        oh_hi, oh_lo = onehots(q)
        z = jnp.dot(lut_d, oh_lo, preferred_element_type=jnp.float32)
        prod = oh_hi * z
        equ = prod[0:HC, :]
        for k in range(1, 16):
            equ = equ + prod[k * HC:(k + 1) * HC, :]
        scale = equ * jnp.float32(1.0 / 255.0)
        f_buf[:, ds, :] = f_buf[:, ds, :] * scale[None, :, :]
        return carry

    jax.lax.fori_loop(0, nch, phase_d, 0)

    cp_out = pltpu.make_async_copy(f_buf, out_hbm.at[b], out_sem)
    cp_out.start()
    cp_out.wait()


@jax.jit
def kernel(F_S):
    B, C, H, W = F_S.shape
    return pl.pallas_call(
        _enhance_body,
        grid=(B,),
        in_specs=[pl.BlockSpec(memory_space=pl.ANY)],
        out_specs=pl.BlockSpec(memory_space=pl.ANY),
        out_shape=jax.ShapeDtypeStruct((B, C, H, W), jnp.float32),
        scratch_shapes=[
            pltpu.VMEM((C, H, W), jnp.float32),
            pltpu.VMEM((H, W), jnp.float32),
            pltpu.SemaphoreType.DMA,
            pltpu.SemaphoreType.DMA,
        ],
        compiler_params=pltpu.CompilerParams(
            dimension_semantics=("parallel",),
            vmem_limit_bytes=100 * 1024 * 1024,
        ),
    )(F_S)


# speedup vs baseline: 49.7210x; 1.8293x over previous
import jax
import jax.numpy as jnp
from jax.experimental import pallas as pl
from jax.experimental.pallas import tpu as pltpu

EPSV = 1e-12
HC = 16
HIGH = jax.lax.Precision.HIGHEST


def _enhance_body(f_hbm, out_hbm, f_bufs, q_ref, in_sems, out_sem):
    p0 = pl.program_id(0)
    j = pl.program_id(1)
    per = pl.num_programs(1)
    b = p0 * per + j
    _, C, H, W = f_bufs.shape
    nch = H // HC
    fP = jnp.float32(H * W)
    cur = jax.lax.rem(j, 2)
    nxt = jax.lax.rem(j + 1, 2)
    x_ref = f_bufs.at[cur]

    cp_in = pltpu.make_async_copy(f_hbm.at[b], x_ref, in_sems.at[cur])

    @pl.when(j == 0)
    def _():
        cp_in.start()

    cp_in.wait()

    def phase_a(i, acc):
        fc = x_ref[:, pl.ds(i * HC, HC), :]
        return acc + jnp.sum(fc, axis=1)

    acc = jax.lax.fori_loop(0, nch, phase_a, jnp.zeros((C, W), jnp.float32))
    a_rep = jnp.dot(acc, jnp.ones((W, W), jnp.float32),
                    precision=HIGH, preferred_element_type=jnp.float32) / fP
    na = jnp.maximum(
        jnp.sqrt(jnp.sum(a_rep * a_rep, axis=0, keepdims=True)), EPSV)
    an2 = a_rep / na

    kpat = jax.lax.broadcasted_iota(jnp.int32, (16 * HC, W), 0) // HC

    def onehots(q):
        hi_t = jnp.tile(q >> 4, (16, 1))
        lo_t = jnp.tile(q & 15, (16, 1))
        oh_hi = jnp.where(hi_t == kpat, 1.0, 0.0)
        oh_lo = jnp.where(lo_t == kpat, 1.0, 0.0)
        return oh_hi, oh_lo

    def phase_b(i, m2):
        ds = pl.ds(i * HC, HC)
        fc = x_ref[:, ds, :]
        dotc = jnp.sum(fc * an2[:, None, :], axis=0)
        ssq = jnp.sum(fc * fc, axis=0)
        npx = jnp.maximum(jnp.sqrt(ssq), EPSV)
        cos = dotc / npx
        qi = (cos * 255.0).astype(jnp.int32)
        q = (qi + 256) & 255
        q_ref[ds, :] = q.astype(jnp.float32)
        oh_hi, oh_lo = onehots(q)
        return m2 + jax.lax.dot_general(
            oh_hi, oh_lo, (((1,), (1,)), ((), ())),
            preferred_element_type=jnp.float32)

    m2 = jax.lax.fori_loop(0, nch, phase_b,
                           jnp.zeros((16 * HC, 16 * HC), jnp.float32))

    @pl.when(j > 0)
    def _():
        pltpu.make_async_copy(f_bufs.at[nxt], out_hbm.at[b - 1],
                              out_sem).wait()

    @pl.when(j < per - 1)
    def _():
        pltpu.make_async_copy(f_hbm.at[b + 1], f_bufs.at[nxt],
                              in_sems.at[nxt]).start()

    n2 = 16 * HC
    i2r = jax.lax.broadcasted_iota(jnp.int32, (n2, n2), 0)
    i2c = jax.lax.broadcasted_iota(jnp.int32, (n2, n2), 1)
    dmask = jnp.where(jax.lax.rem(i2r, HC) == jax.lax.rem(i2c, HC), 1.0, 0.0)
    s16r = jax.lax.broadcasted_iota(jnp.int32, (16, n2), 0)
    s16c = jax.lax.broadcasted_iota(jnp.int32, (16, n2), 1)
    smat = jnp.where(s16r == s16c // HC, 1.0, 0.0)
    hist = jnp.dot(jnp.dot(smat, m2 * dmask, precision=HIGH,
                           preferred_element_type=jnp.float32),
                   smat.T, precision=HIGH,
                   preferred_element_type=jnp.float32)

    r16 = jax.lax.broadcasted_iota(jnp.int32, (16, 16), 0)
    c16 = jax.lax.broadcasted_iota(jnp.int32, (16, 16), 1)
    upper = jnp.where(r16 <= c16, 1.0, 0.0)
    lstrict = jnp.where(c16 < r16, 1.0, 0.0)
    cdf_lo = jnp.dot(hist, upper, precision=HIGH,
                     preferred_element_type=jnp.float32)
    rowsum = jnp.sum(hist, axis=1, keepdims=True)
    offs = jnp.dot(lstrict, rowsum, precision=HIGH,
                   preferred_element_type=jnp.float32)
    cdf = cdf_lo + offs
    masked = jnp.where(hist > 0.0, cdf, fP + 1.0)
    cmin = jnp.min(jnp.min(masked, axis=1, keepdims=True),
                   axis=0, keepdims=True)
    denom = jnp.maximum(fP - cmin, 1.0)
    lut = jnp.clip(jnp.round((cdf - cmin) * (255.0 / denom)), 0.0, 255.0)

    lut_d = jnp.dot(jnp.dot(smat.T, lut,
                            preferred_element_type=jnp.float32),
                    smat, preferred_element_type=jnp.float32) * dmask

    def phase_d(i, carry):
        ds = pl.ds(i * HC, HC)
        q = q_ref[ds, :].astype(jnp.int32)
        oh_hi, oh_lo = onehots(q)
        z = jnp.dot(lut_d, oh_lo, preferred_element_type=jnp.float32)
        prod = oh_hi * z
        equ = prod[0:HC, :]
        for k in range(1, 16):
            equ = equ + prod[k * HC:(k + 1) * HC, :]
        scale = equ * jnp.float32(1.0 / 255.0)
        x_ref[:, ds, :] = x_ref[:, ds, :] * scale[None, :, :]
        return carry

    jax.lax.fori_loop(0, nch, phase_d, 0)

    cp_out = pltpu.make_async_copy(x_ref, out_hbm.at[b], out_sem)
    cp_out.start()

    @pl.when(j == per - 1)
    def _():
        cp_out.wait()


@jax.jit
def kernel(F_S):
    B, C, H, W = F_S.shape
    return pl.pallas_call(
        _enhance_body,
        grid=(2, B // 2),
        in_specs=[pl.BlockSpec(memory_space=pl.ANY)],
        out_specs=pl.BlockSpec(memory_space=pl.ANY),
        out_shape=jax.ShapeDtypeStruct((B, C, H, W), jnp.float32),
        scratch_shapes=[
            pltpu.VMEM((2, C, H, W), jnp.float32),
            pltpu.VMEM((H, W), jnp.float32),
            pltpu.SemaphoreType.DMA((2,)),
            pltpu.SemaphoreType.DMA,
        ],
        compiler_params=pltpu.CompilerParams(
            dimension_semantics=("parallel", "arbitrary"),
            vmem_limit_bytes=100 * 1024 * 1024,
        ),
    )(F_S)
